# baseline (speedup 1.0000x reference)
import functools

import jax
import jax.numpy as jnp
from jax import lax
from jax.experimental import pallas as pl
from jax.experimental.pallas import tpu as pltpu
from jax.experimental.pallas import tpu_sc as plsc

NUM_EMB = 100000
EMB_DIM = 32
BATCH = 16384

_INFO = plsc.get_sparse_core_info()
_NC = _INFO.num_cores
_NS = _INFO.num_subcores
_NW = _NC * _NS
_B_PER_W = BATCH // _NW
_GCHUNK = 128
_NGATHER = _B_PER_W // _GCHUNK


@functools.partial(
    pl.kernel,
    mesh=plsc.VectorSubcoreMesh(core_axis_name="c", subcore_axis_name="s"),
    out_type=jax.ShapeDtypeStruct((EMB_DIM, BATCH), jnp.float32),
    scratch_types=[
        pltpu.VMEM((_B_PER_W,), jnp.int32),
        pltpu.VMEM((_NGATHER, _GCHUNK), jnp.int32),
        pltpu.VMEM((_B_PER_W, 128), jnp.float32),
        pltpu.VMEM((EMB_DIM, _B_PER_W), jnp.float32),
        pltpu.SemaphoreType.DMA,
    ],
    compiler_params=pltpu.CompilerParams(needs_layout_passes=False),
)
def _embed_lookup(idx_hbm, tab_hbm, out_hbm, idx_v, q_v, rows_v, outt_v, sem):
    wid = lax.axis_index("s") * _NC + lax.axis_index("c")
    base = wid * _B_PER_W
    pltpu.sync_copy(idx_hbm.at[pl.ds(base, _B_PER_W)], idx_v)

    for k in range(_B_PER_W // 16):
        v = idx_v[pl.ds(16 * k, 16)]
        q_v[k // 8, pl.ds(16 * (k % 8), 16)] = lax.shift_right_logical(v, 2)

    copies = [
        pltpu.async_copy(
            tab_hbm.at[q_v.at[j]], rows_v.at[pl.ds(_GCHUNK * j, _GCHUNK)], sem
        )
        for j in range(_NGATHER)
    ]
    for cp in copies:
        cp.wait()

    for k in range(_B_PER_W // 16):
        bb = 16 * k
        vidx = idx_v[pl.ds(bb, 16)]
        off = (vidx & 3) * 32
        row_ids = bb + lax.iota(jnp.int32, 16)
        for d in range(EMB_DIM):
            x = plsc.load_gather(rows_v, [row_ids, off + d])
            outt_v[d, pl.ds(bb, 16)] = x

    pltpu.sync_copy(outt_v, out_hbm.at[:, pl.ds(base, _B_PER_W)])


def kernel(char_indices, table):
    tab = table.reshape(NUM_EMB * EMB_DIM // 128, 128)
    out_t = _embed_lookup(char_indices.astype(jnp.int32), tab)
    return out_t.T
